# Initial kernel scaffold; baseline (speedup 1.0000x reference)
import jax
import jax.numpy as jnp
from jax.experimental import pallas as pl


def kernel(bpe_token_ids, table, W_proj, b_proj, ln_gamma, ln_beta):
    raise NotImplementedError("write your pallas kernel here")


# speedup vs baseline: 1.2589x; 1.2589x over previous
import functools

import jax
import jax.numpy as jnp
from jax import lax
from jax.experimental import pallas as pl
from jax.experimental.pallas import tpu as pltpu
from jax.experimental.pallas import tpu_sc as plsc

BATCH = 16384
SUBTOK = 8
PRETRAINED_DIM = 1024
D_MODEL = 512

NC = 2
NS = 16
L = 16
NW = NC * NS

TOK_PER_W = BATCH // NW
T = 4
CH = SUBTOK * T
C = TOK_PER_W // T


def _sc_body(ids_hbm, table_hbm, out_hbm, idx_v, rows0, rows1, fused_v,
             sem0, sem1):
    cid = lax.axis_index("c")
    sid = lax.axis_index("s")
    wid = sid * NC + cid

    pltpu.sync_copy(ids_hbm.at[wid], idx_v)

    rows = (rows0, rows1)
    sems = (sem0, sem1)

    def issue(g, b):
        pltpu.async_copy(table_hbm.at[idx_v.at[g]], rows[b], sems[b])

    issue(0, 0)
    issue(1, 1)

    def accumulate(rows_b, base_row):
        def col_body(kk, _):
            col = pl.ds(kk * L, L)
            for t in range(T):
                acc = rows_b[SUBTOK * t, col]
                for j in range(1, SUBTOK):
                    acc = acc + rows_b[SUBTOK * t + j, col]
                fused_v[t, col] = acc * (1.0 / SUBTOK)
            return 0

        lax.fori_loop(0, PRETRAINED_DIM // L, col_body, 0)
        pltpu.sync_copy(fused_v, out_hbm.at[pl.ds(base_row, T)])

    def chunk_pair(g, _):
        for b in range(2):
            gc = g + b
            pltpu.make_async_copy(table_hbm.at[idx_v.at[gc]], rows[b],
                                  sems[b]).wait()
            accumulate(rows[b], wid * TOK_PER_W + gc * T)

            @pl.when(gc + 2 < C)
            def _():
                issue(gc + 2, b)
        return 0

    lax.fori_loop(0, C // 2, lambda i, c: chunk_pair(i * 2, c), 0)


def _sc_gather_mean(ids, table):
    mesh = plsc.VectorSubcoreMesh(core_axis_name="c", subcore_axis_name="s")
    kern = pl.kernel(
        _sc_body,
        out_type=jax.ShapeDtypeStruct((BATCH, PRETRAINED_DIM), jnp.float32),
        mesh=mesh,
        scratch_types=[
            pltpu.VMEM((C, CH), jnp.int32),
            pltpu.VMEM((CH, PRETRAINED_DIM), jnp.float32),
            pltpu.VMEM((CH, PRETRAINED_DIM), jnp.float32),
            pltpu.VMEM((T, PRETRAINED_DIM), jnp.float32),
            pltpu.SemaphoreType.DMA,
            pltpu.SemaphoreType.DMA,
        ],
    )
    return kern(ids, table)


def _tc_body(fused_ref, w_ref, b_ref, g_ref, beta_ref, out_ref):
    x = jnp.dot(fused_ref[...], w_ref[...],
                preferred_element_type=jnp.float32)
    x = x + b_ref[...]
    mu = jnp.mean(x, axis=-1, keepdims=True)
    xc = x - mu
    var = jnp.mean(xc * xc, axis=-1, keepdims=True)
    out_ref[...] = g_ref[...] * (xc * lax.rsqrt(var + 1e-5)) + beta_ref[...]


def _tc_proj_ln(fused, W_proj, b_proj, ln_gamma, ln_beta):
    BM = 1024
    grid = (BATCH // BM,)
    return pl.pallas_call(
        _tc_body,
        grid=grid,
        in_specs=[
            pl.BlockSpec((BM, PRETRAINED_DIM), lambda i: (i, 0)),
            pl.BlockSpec((PRETRAINED_DIM, D_MODEL), lambda i: (0, 0)),
            pl.BlockSpec((1, D_MODEL), lambda i: (0, 0)),
            pl.BlockSpec((1, D_MODEL), lambda i: (0, 0)),
            pl.BlockSpec((1, D_MODEL), lambda i: (0, 0)),
        ],
        out_specs=pl.BlockSpec((BM, D_MODEL), lambda i: (i, 0)),
        out_shape=jax.ShapeDtypeStruct((BATCH, D_MODEL), jnp.float32),
    )(fused, W_proj, b_proj.reshape(1, D_MODEL),
      ln_gamma.reshape(1, D_MODEL), ln_beta.reshape(1, D_MODEL))


def kernel(bpe_token_ids, table, W_proj, b_proj, ln_gamma, ln_beta):
    ids = bpe_token_ids.astype(jnp.int32).reshape(NW, C, CH)
    fused = _sc_gather_mean(ids, table)
    return _tc_proj_ln(fused, W_proj, b_proj, ln_gamma, ln_beta)


# speedup vs baseline: 1.3930x; 1.1065x over previous
import jax
import jax.numpy as jnp
from jax import lax
from jax.experimental import pallas as pl
from jax.experimental.pallas import tpu as pltpu
from jax.experimental.pallas import tpu_sc as plsc

BATCH = 16384
SUBTOK = 8
PRETRAINED_DIM = 1024
D_MODEL = 512

NC = 2
NS = 16
L = 16
NW = NC * NS

TOK_PER_W = BATCH // NW
T = 4
CH = SUBTOK * T
C = TOK_PER_W // T


def _sc_body(ids_hbm, table_hbm, out_hbm, idx_v, rows0, rows1, fused_v,
             sem0, sem1):
    cid = lax.axis_index("c")
    sid = lax.axis_index("s")
    wid = sid * NC + cid

    pltpu.sync_copy(ids_hbm.at[wid], idx_v)

    rows = (rows0, rows1)
    sems = (sem0, sem1)

    def issue(g, b):
        pltpu.async_copy(table_hbm.at[idx_v.at[g]], rows[b], sems[b])

    issue(0, 0)
    issue(1, 1)

    def accumulate(rows_b, base_row):
        def col_body(kk, _):
            col = pl.ds(kk * L, L)
            for t in range(T):
                v = [rows_b[SUBTOK * t + j, col] for j in range(SUBTOK)]
                while len(v) > 1:
                    v = [v[2 * i] + v[2 * i + 1] for i in range(len(v) // 2)]
                fused_v[t, col] = v[0]
            return 0

        lax.fori_loop(0, PRETRAINED_DIM // L, col_body, 0, unroll=2)
        pltpu.sync_copy(fused_v, out_hbm.at[pl.ds(base_row, T)])

    def chunk_pair(g, _):
        for b in range(2):
            gc = g + b
            pltpu.make_async_copy(table_hbm.at[idx_v.at[gc]], rows[b],
                                  sems[b]).wait()
            accumulate(rows[b], wid * TOK_PER_W + gc * T)

            @pl.when(gc + 2 < C)
            def _():
                issue(gc + 2, b)
        return 0

    lax.fori_loop(0, C // 2, lambda i, c: chunk_pair(i * 2, c), 0)


def _sc_gather_sum(ids, table):
    mesh = plsc.VectorSubcoreMesh(core_axis_name="c", subcore_axis_name="s")
    kern = pl.kernel(
        _sc_body,
        out_type=jax.ShapeDtypeStruct((BATCH, PRETRAINED_DIM), jnp.float32),
        mesh=mesh,
        scratch_types=[
            pltpu.VMEM((C, CH), jnp.int32),
            pltpu.VMEM((CH, PRETRAINED_DIM), jnp.float32),
            pltpu.VMEM((CH, PRETRAINED_DIM), jnp.float32),
            pltpu.VMEM((T, PRETRAINED_DIM), jnp.float32),
            pltpu.SemaphoreType.DMA,
            pltpu.SemaphoreType.DMA,
        ],
    )
    return kern(ids, table)


def _tc_body(fused_ref, w_ref, b_ref, g_ref, beta_ref, out_ref):
    x = jnp.dot(fused_ref[...], w_ref[...],
                preferred_element_type=jnp.float32)
    x = x * (1.0 / SUBTOK) + b_ref[...]
    mu = jnp.mean(x, axis=-1, keepdims=True)
    xc = x - mu
    var = jnp.mean(xc * xc, axis=-1, keepdims=True)
    out_ref[...] = g_ref[...] * (xc * lax.rsqrt(var + 1e-5)) + beta_ref[...]


def _tc_proj_ln(fused, W_proj, b_proj, ln_gamma, ln_beta):
    BM = 1024
    grid = (BATCH // BM,)
    return pl.pallas_call(
        _tc_body,
        grid=grid,
        in_specs=[
            pl.BlockSpec((BM, PRETRAINED_DIM), lambda i: (i, 0)),
            pl.BlockSpec((PRETRAINED_DIM, D_MODEL), lambda i: (0, 0)),
            pl.BlockSpec((1, D_MODEL), lambda i: (0, 0)),
            pl.BlockSpec((1, D_MODEL), lambda i: (0, 0)),
            pl.BlockSpec((1, D_MODEL), lambda i: (0, 0)),
        ],
        out_specs=pl.BlockSpec((BM, D_MODEL), lambda i: (i, 0)),
        out_shape=jax.ShapeDtypeStruct((BATCH, D_MODEL), jnp.float32),
    )(fused, W_proj, b_proj.reshape(1, D_MODEL),
      ln_gamma.reshape(1, D_MODEL), ln_beta.reshape(1, D_MODEL))


def kernel(bpe_token_ids, table, W_proj, b_proj, ln_gamma, ln_beta):
    ids = bpe_token_ids.astype(jnp.int32).reshape(NW, C, CH)
    fused = _sc_gather_sum(ids, table)
    return _tc_proj_ln(fused, W_proj, b_proj, ln_gamma, ln_beta)


# speedup vs baseline: 1.5605x; 1.1203x over previous
import jax
import jax.numpy as jnp
from jax import lax
from jax.experimental import pallas as pl
from jax.experimental.pallas import tpu as pltpu
from jax.experimental.pallas import tpu_sc as plsc

BATCH = 16384
SUBTOK = 8
PRETRAINED_DIM = 1024
D_MODEL = 512

NC = 2
NS = 16
L = 16
NW = NC * NS

TOK_PER_W = BATCH // NW
T = 2
CH = SUBTOK * T
C = TOK_PER_W // T
NBUF = 4
NFB = 2


def _sc_body(ids_hbm, table_hbm, out_hbm, idx_v,
             rows0, rows1, rows2, rows3, fused0, fused1,
             sem0, sem1, sem2, sem3, osem0, osem1):
    cid = lax.axis_index("c")
    sid = lax.axis_index("s")
    wid = sid * NC + cid

    pltpu.sync_copy(ids_hbm.at[wid], idx_v)

    rows = (rows0, rows1, rows2, rows3)
    sems = (sem0, sem1, sem2, sem3)
    fused = (fused0, fused1)
    osems = (osem0, osem1)

    def issue(g, b):
        pltpu.async_copy(table_hbm.at[idx_v.at[g]], rows[b], sems[b])

    for b in range(NBUF):
        issue(b, b)

    def accumulate(rows_b, fused_f):
        def col_body(kk, _):
            col = pl.ds(kk * L, L)
            for t in range(T):
                v = [rows_b[SUBTOK * t + j, col] for j in range(SUBTOK)]
                while len(v) > 1:
                    v = [v[2 * i] + v[2 * i + 1] for i in range(len(v) // 2)]
                fused_f[t, col] = v[0]
            return 0

        lax.fori_loop(0, PRETRAINED_DIM // L, col_body, 0, unroll=2)

    def chunk(gc, b, f):
        pltpu.make_async_copy(table_hbm.at[idx_v.at[gc]], rows[b],
                              sems[b]).wait()

        @pl.when(gc >= NFB)
        def _():
            pltpu.make_async_copy(fused[f], out_hbm.at[pl.ds(0, T)],
                                  osems[f]).wait()

        accumulate(rows[b], fused[f])
        base = wid * TOK_PER_W + gc * T
        pltpu.async_copy(fused[f], out_hbm.at[pl.ds(base, T)], osems[f])

        @pl.when(gc + NBUF < C)
        def _():
            issue(gc + NBUF, b)

    def chunk_quad(g, _):
        for b in range(NBUF):
            chunk(g + b, b, b % NFB)
        return 0

    lax.fori_loop(0, C // NBUF, lambda i, c: chunk_quad(i * NBUF, c), 0)

    for f in range(NFB):
        pltpu.make_async_copy(fused[f], out_hbm.at[pl.ds(0, T)],
                              osems[f]).wait()


def _sc_gather_sum(ids, table):
    mesh = plsc.VectorSubcoreMesh(core_axis_name="c", subcore_axis_name="s")
    kern = pl.kernel(
        _sc_body,
        out_type=jax.ShapeDtypeStruct((BATCH, PRETRAINED_DIM), jnp.float32),
        mesh=mesh,
        scratch_types=(
            [pltpu.VMEM((C, CH), jnp.int32)]
            + [pltpu.VMEM((CH, PRETRAINED_DIM), jnp.float32)
               for _ in range(NBUF)]
            + [pltpu.VMEM((T, PRETRAINED_DIM), jnp.float32)
               for _ in range(NFB)]
            + [pltpu.SemaphoreType.DMA for _ in range(NBUF + NFB)]
        ),
    )
    return kern(ids, table)


def _tc_body(fused_ref, w_ref, b_ref, g_ref, beta_ref, out_ref):
    x = jnp.dot(fused_ref[...], w_ref[...],
                preferred_element_type=jnp.float32)
    x = x * (1.0 / SUBTOK) + b_ref[...]
    mu = jnp.mean(x, axis=-1, keepdims=True)
    xc = x - mu
    var = jnp.mean(xc * xc, axis=-1, keepdims=True)
    out_ref[...] = g_ref[...] * (xc * lax.rsqrt(var + 1e-5)) + beta_ref[...]


def _tc_proj_ln(fused, W_proj, b_proj, ln_gamma, ln_beta):
    BM = 1024
    grid = (BATCH // BM,)
    return pl.pallas_call(
        _tc_body,
        grid=grid,
        in_specs=[
            pl.BlockSpec((BM, PRETRAINED_DIM), lambda i: (i, 0)),
            pl.BlockSpec((PRETRAINED_DIM, D_MODEL), lambda i: (0, 0)),
            pl.BlockSpec((1, D_MODEL), lambda i: (0, 0)),
            pl.BlockSpec((1, D_MODEL), lambda i: (0, 0)),
            pl.BlockSpec((1, D_MODEL), lambda i: (0, 0)),
        ],
        out_specs=pl.BlockSpec((BM, D_MODEL), lambda i: (i, 0)),
        out_shape=jax.ShapeDtypeStruct((BATCH, D_MODEL), jnp.float32),
    )(fused, W_proj, b_proj.reshape(1, D_MODEL),
      ln_gamma.reshape(1, D_MODEL), ln_beta.reshape(1, D_MODEL))


def kernel(bpe_token_ids, table, W_proj, b_proj, ln_gamma, ln_beta):
    ids = bpe_token_ids.astype(jnp.int32).reshape(NW, C, CH)
    fused = _sc_gather_sum(ids, table)
    return _tc_proj_ln(fused, W_proj, b_proj, ln_gamma, ln_beta)


# speedup vs baseline: 2.4761x; 1.5867x over previous
import jax
import jax.numpy as jnp
from jax import lax
from jax.experimental import pallas as pl
from jax.experimental.pallas import tpu as pltpu
from jax.experimental.pallas import tpu_sc as plsc

BATCH = 16384
SUBTOK = 8
PRETRAINED_DIM = 1024
D_MODEL = 512

NC = 2
NS = 16
L = 16
NW = NC * NS

TOK_PER_W = BATCH // NW
T = 2
CH = SUBTOK * T
C = TOK_PER_W // T
NBUF = 4
NFB = 2


def _sc_body(ids_hbm, table_hbm, out_hbm, idx_v,
             rows0, rows1, rows2, rows3, fused0, fused1,
             sem0, sem1, sem2, sem3, osem0, osem1):
    cid = lax.axis_index("c")
    sid = lax.axis_index("s")
    wid = sid * NC + cid

    pltpu.sync_copy(ids_hbm.at[wid], idx_v)

    rows = (rows0, rows1, rows2, rows3)
    sems = (sem0, sem1, sem2, sem3)
    fused = (fused0, fused1)
    osems = (osem0, osem1)

    def issue(g, b):
        pltpu.async_copy(table_hbm.at[idx_v.at[g]], rows[b], sems[b])

    for b in range(NBUF):
        issue(b, b)

    def accumulate(rows_b, fused_f):
        @plsc.parallel_loop(0, PRETRAINED_DIM // L, step=1, unroll=4)
        def col_body(kk):
            col = pl.ds(kk * L, L)
            for t in range(T):
                v = [rows_b[SUBTOK * t + j, col] for j in range(SUBTOK)]
                while len(v) > 1:
                    v = [v[2 * i] + v[2 * i + 1] for i in range(len(v) // 2)]
                fused_f[t, col] = v[0]

    def chunk(gc, b, f):
        pltpu.make_async_copy(table_hbm.at[idx_v.at[gc]], rows[b],
                              sems[b]).wait()

        @pl.when(gc >= NFB)
        def _():
            pltpu.make_async_copy(fused[f], out_hbm.at[pl.ds(0, T)],
                                  osems[f]).wait()

        accumulate(rows[b], fused[f])
        base = wid * TOK_PER_W + gc * T
        pltpu.async_copy(fused[f], out_hbm.at[pl.ds(base, T)], osems[f])

        @pl.when(gc + NBUF < C)
        def _():
            issue(gc + NBUF, b)

    def chunk_quad(g, _):
        for b in range(NBUF):
            chunk(g + b, b, b % NFB)
        return 0

    lax.fori_loop(0, C // NBUF, lambda i, c: chunk_quad(i * NBUF, c), 0)

    for f in range(NFB):
        pltpu.make_async_copy(fused[f], out_hbm.at[pl.ds(0, T)],
                              osems[f]).wait()


def _sc_gather_sum(ids, table):
    mesh = plsc.VectorSubcoreMesh(core_axis_name="c", subcore_axis_name="s")
    kern = pl.kernel(
        _sc_body,
        out_type=jax.ShapeDtypeStruct((BATCH, PRETRAINED_DIM), jnp.float32),
        mesh=mesh,
        scratch_types=(
            [pltpu.VMEM((C, CH), jnp.int32)]
            + [pltpu.VMEM((CH, PRETRAINED_DIM), jnp.float32)
               for _ in range(NBUF)]
            + [pltpu.VMEM((T, PRETRAINED_DIM), jnp.float32)
               for _ in range(NFB)]
            + [pltpu.SemaphoreType.DMA for _ in range(NBUF + NFB)]
        ),
    )
    return kern(ids, table)


def _tc_body(fused_ref, w_ref, b_ref, g_ref, beta_ref, out_ref):
    x = jnp.dot(fused_ref[...], w_ref[...],
                preferred_element_type=jnp.float32)
    x = x * (1.0 / SUBTOK) + b_ref[...]
    mu = jnp.mean(x, axis=-1, keepdims=True)
    xc = x - mu
    var = jnp.mean(xc * xc, axis=-1, keepdims=True)
    out_ref[...] = g_ref[...] * (xc * lax.rsqrt(var + 1e-5)) + beta_ref[...]


def _tc_proj_ln(fused, W_proj, b_proj, ln_gamma, ln_beta):
    BM = 1024
    grid = (BATCH // BM,)
    return pl.pallas_call(
        _tc_body,
        grid=grid,
        in_specs=[
            pl.BlockSpec((BM, PRETRAINED_DIM), lambda i: (i, 0)),
            pl.BlockSpec((PRETRAINED_DIM, D_MODEL), lambda i: (0, 0)),
            pl.BlockSpec((1, D_MODEL), lambda i: (0, 0)),
            pl.BlockSpec((1, D_MODEL), lambda i: (0, 0)),
            pl.BlockSpec((1, D_MODEL), lambda i: (0, 0)),
        ],
        out_specs=pl.BlockSpec((BM, D_MODEL), lambda i: (i, 0)),
        out_shape=jax.ShapeDtypeStruct((BATCH, D_MODEL), jnp.float32),
    )(fused, W_proj, b_proj.reshape(1, D_MODEL),
      ln_gamma.reshape(1, D_MODEL), ln_beta.reshape(1, D_MODEL))


def kernel(bpe_token_ids, table, W_proj, b_proj, ln_gamma, ln_beta):
    ids = bpe_token_ids.astype(jnp.int32).reshape(NW, C, CH)
    fused = _sc_gather_sum(ids, table)
    return _tc_proj_ln(fused, W_proj, b_proj, ln_gamma, ln_beta)


# speedup vs baseline: 2.4812x; 1.0021x over previous
import jax
import jax.numpy as jnp
from jax import lax
from jax.experimental import pallas as pl
from jax.experimental.pallas import tpu as pltpu
from jax.experimental.pallas import tpu_sc as plsc

BATCH = 16384
SUBTOK = 8
PRETRAINED_DIM = 1024
D_MODEL = 512

NC = 2
NS = 16
L = 16
NW = NC * NS

TOK_PER_W = BATCH // NW
T = 2
CH = SUBTOK * T
C = TOK_PER_W // T
NBUF = 4
NFB = 2


def _sc_body(ids_hbm, table_hbm, out_hbm, idx_v,
             rows0, rows1, rows2, rows3, fused0, fused1,
             sem0, sem1, sem2, sem3, osem0, osem1):
    cid = lax.axis_index("c")
    sid = lax.axis_index("s")
    wid = sid * NC + cid

    pltpu.sync_copy(ids_hbm.at[wid], idx_v)

    rows = (rows0, rows1, rows2, rows3)
    sems = (sem0, sem1, sem2, sem3)
    fused = (fused0, fused1)
    osems = (osem0, osem1)

    def issue(g, b):
        pltpu.async_copy(table_hbm.at[idx_v.at[g]], rows[b], sems[b])

    for b in range(NBUF):
        issue(b, b)

    def accumulate(rows_b, fused_f):
        @plsc.parallel_loop(0, PRETRAINED_DIM // L, step=1, unroll=8)
        def col_body(kk):
            col = pl.ds(kk * L, L)
            for t in range(T):
                v = [rows_b[SUBTOK * t + j, col] for j in range(SUBTOK)]
                while len(v) > 1:
                    v = [v[2 * i] + v[2 * i + 1] for i in range(len(v) // 2)]
                fused_f[t, col] = v[0]

    def chunk(gc, b, f):
        pltpu.make_async_copy(table_hbm.at[idx_v.at[gc]], rows[b],
                              sems[b]).wait()

        @pl.when(gc >= NFB)
        def _():
            pltpu.make_async_copy(fused[f], out_hbm.at[pl.ds(0, T)],
                                  osems[f]).wait()

        accumulate(rows[b], fused[f])
        base = wid * TOK_PER_W + gc * T
        pltpu.async_copy(fused[f], out_hbm.at[pl.ds(base, T)], osems[f])

        @pl.when(gc + NBUF < C)
        def _():
            issue(gc + NBUF, b)

    def chunk_quad(g, _):
        for b in range(NBUF):
            chunk(g + b, b, b % NFB)
        return 0

    lax.fori_loop(0, C // NBUF, lambda i, c: chunk_quad(i * NBUF, c), 0)

    for f in range(NFB):
        pltpu.make_async_copy(fused[f], out_hbm.at[pl.ds(0, T)],
                              osems[f]).wait()


def _sc_gather_sum(ids, table):
    mesh = plsc.VectorSubcoreMesh(core_axis_name="c", subcore_axis_name="s")
    kern = pl.kernel(
        _sc_body,
        out_type=jax.ShapeDtypeStruct((BATCH, PRETRAINED_DIM), jnp.float32),
        mesh=mesh,
        scratch_types=(
            [pltpu.VMEM((C, CH), jnp.int32)]
            + [pltpu.VMEM((CH, PRETRAINED_DIM), jnp.float32)
               for _ in range(NBUF)]
            + [pltpu.VMEM((T, PRETRAINED_DIM), jnp.float32)
               for _ in range(NFB)]
            + [pltpu.SemaphoreType.DMA for _ in range(NBUF + NFB)]
        ),
    )
    return kern(ids, table)


def _tc_body(fused_ref, w_ref, b_ref, g_ref, beta_ref, out_ref):
    x = jnp.dot(fused_ref[...], w_ref[...],
                preferred_element_type=jnp.float32)
    x = x * (1.0 / SUBTOK) + b_ref[...]
    mu = jnp.mean(x, axis=-1, keepdims=True)
    xc = x - mu
    var = jnp.mean(xc * xc, axis=-1, keepdims=True)
    out_ref[...] = g_ref[...] * (xc * lax.rsqrt(var + 1e-5)) + beta_ref[...]


def _tc_proj_ln(fused, W_proj, b_proj, ln_gamma, ln_beta):
    BM = 1024
    grid = (BATCH // BM,)
    return pl.pallas_call(
        _tc_body,
        grid=grid,
        in_specs=[
            pl.BlockSpec((BM, PRETRAINED_DIM), lambda i: (i, 0)),
            pl.BlockSpec((PRETRAINED_DIM, D_MODEL), lambda i: (0, 0)),
            pl.BlockSpec((1, D_MODEL), lambda i: (0, 0)),
            pl.BlockSpec((1, D_MODEL), lambda i: (0, 0)),
            pl.BlockSpec((1, D_MODEL), lambda i: (0, 0)),
        ],
        out_specs=pl.BlockSpec((BM, D_MODEL), lambda i: (i, 0)),
        out_shape=jax.ShapeDtypeStruct((BATCH, D_MODEL), jnp.float32),
    )(fused, W_proj, b_proj.reshape(1, D_MODEL),
      ln_gamma.reshape(1, D_MODEL), ln_beta.reshape(1, D_MODEL))


def kernel(bpe_token_ids, table, W_proj, b_proj, ln_gamma, ln_beta):
    ids = bpe_token_ids.astype(jnp.int32).reshape(NW, C, CH)
    fused = _sc_gather_sum(ids, table)
    return _tc_proj_ln(fused, W_proj, b_proj, ln_gamma, ln_beta)
